# baseline (device time: 137394 ns/iter reference)
import jax
import jax.numpy as jnp
from jax import lax
from jax.experimental import pallas as pl
from jax.experimental.pallas import tpu as pltpu

N_DEV = 8
NBR = [
    [1, 0, 3, 2, 5, 4, 7, 6],
    [3, 2, 1, 0, 7, 6, 5, 4],
    [4, 5, 6, 7, 0, 1, 2, 3],
]
SLOT = [
    [0, 1, 3, 2, 4, 5, 7, 6],
    [0, 4, 5, 1, 2, 6, 7, 3],
    [0, 2, 6, 4, 1, 3, 7, 5],
]
OTAB = [
    [0, 1, 3, 2, 4, 5, 7, 6],
    [0, 3, 4, 7, 1, 2, 5, 6],
    [0, 4, 1, 5, 3, 7, 2, 6],
]
ROWS = [176, 168, 168]
ROW_OFF = [0, 176, 344]


def _sel(idx, table):
    acc = jnp.int32(0)
    for p, v in enumerate(table):
        acc = acc + jnp.where(idx == p, jnp.int32(v), 0)
    return acc


def kernel(x, w_mat):
    m_per, k = x.shape
    _, n_per = w_mat.shape
    m_total = N_DEV * m_per

    def body(x_ref, w_ref, out_ref, b0, b1, b2, ssems, rsems,
             amax_ref, a_ssems, a_rsems):
        my = lax.axis_index("i")
        bufs = [b0, b1, b2]

        w_f32 = w_ref[:, :]
        w_hi = w_f32.astype(jnp.bfloat16)
        w_lo = (w_f32 - w_hi.astype(jnp.float32)).astype(jnp.bfloat16)

        def mm(a):
            return (
                jnp.dot(a, w_hi, preferred_element_type=jnp.float32)
                + jnp.dot(a, w_lo, preferred_element_type=jnp.float32)
            )

        amax_parts = []

        def part_matmul(r, s_dyn):
            o = _sel(s_dyn, OTAB[r])
            y = mm(bufs[r][pl.ds(s_dyn * ROWS[r], ROWS[r]), :])
            out_ref[pl.ds(o * m_per + ROW_OFF[r], ROWS[r]), :] = y
            amax_parts.append(jnp.max(jnp.abs(y)))

        barrier_sem = pltpu.get_barrier_semaphore()
        nbrs = [_sel(my, NBR[d]) for d in range(3)]
        for nbr in nbrs:
            pl.semaphore_signal(
                barrier_sem, inc=1,
                device_id=(nbr,), device_id_type=pl.DeviceIdType.MESH,
            )
        pl.semaphore_wait(barrier_sem, 3)

        slots = [_sel(my, SLOT[r]) for r in range(3)]

        for t in range(3):
            size = 1 << t
            waits = []
            for r in range(3):
                nbr = nbrs[(r + t) % 3]
                base = (slots[r] // size) * size
                if t == 0:
                    bufs[r][pl.ds(slots[r] * ROWS[r], ROWS[r]), :] = (
                        x_ref[ROW_OFF[r]:ROW_OFF[r] + ROWS[r], :]
                        .astype(jnp.bfloat16)
                    )
                if t < 2:
                    rdma = pltpu.make_async_remote_copy(
                        src_ref=bufs[r].at[pl.ds(base * ROWS[r], size * ROWS[r])],
                        dst_ref=bufs[r].at[pl.ds(base * ROWS[r], size * ROWS[r])],
                        send_sem=ssems.at[r, t], recv_sem=rsems.at[r, t],
                        device_id=(nbr,), device_id_type=pl.DeviceIdType.MESH,
                    )
                    rdma.start()
                    waits.append(rdma)
                else:
                    for j in range(4):
                        rdma = pltpu.make_async_remote_copy(
                            src_ref=bufs[r].at[pl.ds((base + j) * ROWS[r], ROWS[r])],
                            dst_ref=bufs[r].at[pl.ds((base + j) * ROWS[r], ROWS[r])],
                            send_sem=ssems.at[r, 2 + j], recv_sem=rsems.at[r, 2 + j],
                            device_id=(nbr,), device_id_type=pl.DeviceIdType.MESH,
                        )
                        rdma.start()
                        waits.append(rdma)

            if t == 0:
                for r in range(3):
                    part_matmul(r, slots[r])
            elif t == 1:
                for r in range(3):
                    rs = slots[r] + 1 - 2 * (slots[r] % 2)
                    part_matmul(r, rs)
            else:
                for r in range(3):
                    b1bit = (slots[r] // 2) % 2
                    rb = (slots[r] // 4) * 4 + (1 - b1bit) * 2
                    part_matmul(r, rb)
                    part_matmul(r, rb + 1)

            if t < 2:
                for rdma in waits:
                    rdma.wait()
            else:
                for j in range(4):
                    for r in range(3):
                        waits[r * 4 + j].wait()
                    for r in range(3):
                        rbase = (1 - slots[r] // 4) * 4
                        part_matmul(r, rbase + j)

        local_amax = amax_parts[0]
        for a in amax_parts[1:]:
            local_amax = jnp.maximum(local_amax, a)
        amax_ref[0, :, :] = jnp.full((1, 128), local_amax, jnp.float32)
        for t in range(3):
            rdma = pltpu.make_async_remote_copy(
                src_ref=amax_ref.at[0], dst_ref=amax_ref.at[1 + t],
                send_sem=a_ssems.at[t], recv_sem=a_rsems.at[t],
                device_id=(nbrs[t],), device_id_type=pl.DeviceIdType.MESH,
            )
            rdma.start()
            rdma.wait()
            amax_ref[0, :, :] = jnp.maximum(
                amax_ref[0, :, :], amax_ref[1 + t, :, :]
            )

        gmax = jnp.max(amax_ref[0, :, :])
        scale = gmax / 127.0
        y = out_ref[:, :]
        q = jnp.clip(jnp.round(y / scale), -127.0, 127.0)
        out_ref[:, :] = q * scale

    return pl.pallas_call(
        body,
        out_shape=jax.ShapeDtypeStruct((m_total, n_per), jnp.float32),
        in_specs=[
            pl.BlockSpec(memory_space=pltpu.VMEM),
            pl.BlockSpec(memory_space=pltpu.VMEM),
        ],
        out_specs=pl.BlockSpec(memory_space=pltpu.VMEM),
        scratch_shapes=[
            pltpu.VMEM((N_DEV * ROWS[0], k), jnp.bfloat16),
            pltpu.VMEM((N_DEV * ROWS[1], k), jnp.bfloat16),
            pltpu.VMEM((N_DEV * ROWS[2], k), jnp.bfloat16),
            pltpu.SemaphoreType.DMA((3, 6)),
            pltpu.SemaphoreType.DMA((3, 6)),
            pltpu.VMEM((4, 1, 128), jnp.float32),
            pltpu.SemaphoreType.DMA((3,)),
            pltpu.SemaphoreType.DMA((3,)),
        ],
        compiler_params=pltpu.CompilerParams(
            collective_id=0,
            vmem_limit_bytes=56 * 1024 * 1024,
        ),
    )(x, w_mat)


# device time: 90664 ns/iter; 1.5154x vs baseline; 1.5154x over previous
import jax
import jax.numpy as jnp
from jax import lax
from jax.experimental import pallas as pl
from jax.experimental.pallas import tpu as pltpu

N_DEV = 8
NBR = [
    [1, 0, 3, 2, 5, 4, 7, 6],
    [3, 2, 1, 0, 7, 6, 5, 4],
    [4, 5, 6, 7, 0, 1, 2, 3],
]
SLOT = [
    [0, 1, 3, 2, 4, 5, 7, 6],
    [0, 4, 5, 1, 2, 6, 7, 3],
    [0, 2, 6, 4, 1, 3, 7, 5],
]
OTAB = [
    [0, 1, 3, 2, 4, 5, 7, 6],
    [0, 3, 4, 7, 1, 2, 5, 6],
    [0, 4, 1, 5, 3, 7, 2, 6],
]
KROWS = [1376, 1360, 1360]
KOFF = [0, 1376, 2736]


def _sel(idx, table):
    acc = jnp.int32(0)
    for p, v in enumerate(table):
        acc = acc + jnp.where(idx == p, jnp.int32(v), 0)
    return acc


def kernel(x, w_mat):
    m_per, k = x.shape
    _, n_per = w_mat.shape
    m_total = N_DEV * m_per
    n_total = N_DEV * n_per

    def body(x_ref, w_ref, out_ref, b0, b1, b2, ssems, rsems,
             xb_ref, y_ref, q_ref, tiles_ref, t_ssems, t_rsems,
             amax_ref, a_ssems, a_rsems):
        my = lax.axis_index("i")
        bufs = [b0, b1, b2]
        nbrs = [_sel(my, NBR[d]) for d in range(3)]
        slots = [_sel(my, SLOT[r]) for r in range(3)]

        barrier_sem = pltpu.get_barrier_semaphore()
        for joff in range(1, N_DEV):
            pl.semaphore_signal(
                barrier_sem, inc=1,
                device_id=((my + joff) % N_DEV,),
                device_id_type=pl.DeviceIdType.MESH,
            )
        xb_ref[:, :] = x_ref[:, :].astype(jnp.bfloat16)
        y_ref[:, :] = jnp.zeros((m_per, n_total), jnp.float32)
        pl.semaphore_wait(barrier_sem, N_DEV - 1)

        def piece_matmul(r, s_dyn):
            o = _sel(s_dyn, OTAB[r])
            dot = jnp.dot(
                xb_ref[:, KOFF[r]:KOFF[r] + KROWS[r]],
                bufs[r][pl.ds(s_dyn * KROWS[r], KROWS[r]), :],
                preferred_element_type=jnp.float32,
            )
            y_ref[:, pl.ds(o * n_per, n_per)] = (
                y_ref[:, pl.ds(o * n_per, n_per)] + dot
            )

        for t in range(3):
            size = 1 << t
            waits = []
            for r in range(3):
                nbr = nbrs[(r + t) % 3]
                base = (slots[r] // size) * size
                if t == 0:
                    bufs[r][pl.ds(slots[r] * KROWS[r], KROWS[r]), :] = (
                        w_ref[KOFF[r]:KOFF[r] + KROWS[r], :]
                        .astype(jnp.bfloat16)
                    )
                if t < 2:
                    rdma = pltpu.make_async_remote_copy(
                        src_ref=bufs[r].at[pl.ds(base * KROWS[r], size * KROWS[r])],
                        dst_ref=bufs[r].at[pl.ds(base * KROWS[r], size * KROWS[r])],
                        send_sem=ssems.at[r, t], recv_sem=rsems.at[r, t],
                        device_id=(nbr,), device_id_type=pl.DeviceIdType.MESH,
                    )
                    rdma.start()
                    waits.append(rdma)
                else:
                    for j in range(4):
                        rdma = pltpu.make_async_remote_copy(
                            src_ref=bufs[r].at[pl.ds((base + j) * KROWS[r], KROWS[r])],
                            dst_ref=bufs[r].at[pl.ds((base + j) * KROWS[r], KROWS[r])],
                            send_sem=ssems.at[r, 2 + j], recv_sem=rsems.at[r, 2 + j],
                            device_id=(nbr,), device_id_type=pl.DeviceIdType.MESH,
                        )
                        rdma.start()
                        waits.append(rdma)

            if t == 0:
                for r in range(3):
                    piece_matmul(r, slots[r])
            elif t == 1:
                for r in range(3):
                    rs = slots[r] + 1 - 2 * (slots[r] % 2)
                    piece_matmul(r, rs)
            else:
                for r in range(3):
                    b1bit = (slots[r] // 2) % 2
                    rb = (slots[r] // 4) * 4 + (1 - b1bit) * 2
                    piece_matmul(r, rb)
                    piece_matmul(r, rb + 1)

            if t < 2:
                for rdma in waits:
                    rdma.wait()
            else:
                for j in range(4):
                    for r in range(3):
                        waits[r * 4 + j].wait()
                    for r in range(3):
                        rbase = (1 - slots[r] // 4) * 4
                        piece_matmul(r, rbase + j)

        local_amax = jnp.max(jnp.abs(y_ref[:, :]))
        amax_ref[0, :, :] = jnp.full((1, 128), local_amax, jnp.float32)
        for t in range(3):
            rdma = pltpu.make_async_remote_copy(
                src_ref=amax_ref.at[0], dst_ref=amax_ref.at[1 + t],
                send_sem=a_ssems.at[t], recv_sem=a_rsems.at[t],
                device_id=(nbrs[t],), device_id_type=pl.DeviceIdType.MESH,
            )
            rdma.start()
            rdma.wait()
            amax_ref[0, :, :] = jnp.maximum(
                amax_ref[0, :, :], amax_ref[1 + t, :, :]
            )

        gmax = jnp.max(amax_ref[0, :, :])
        scale = gmax / 127.0
        inv = 127.0 / gmax

        for o in range(N_DEV):
            yblk = y_ref[:, o * n_per:(o + 1) * n_per]
            q = jnp.clip(jnp.round(yblk * inv), -127.0, 127.0)
            q_ref[o * m_per:(o + 1) * m_per, :] = q.astype(jnp.int8)

        y_own = y_ref[:, pl.ds(my * n_per, n_per)]
        q_own = jnp.clip(jnp.round(y_own * inv), -127.0, 127.0)
        out_ref[pl.ds(my * m_per, m_per), :] = q_own * scale

        sends = []
        for joff in range(1, N_DEV):
            d = (my + joff) % N_DEV
            rdma = pltpu.make_async_remote_copy(
                src_ref=q_ref.at[pl.ds(d * m_per, m_per)],
                dst_ref=tiles_ref.at[pl.ds(my * m_per, m_per)],
                send_sem=t_ssems.at[joff - 1],
                recv_sem=t_rsems.at[joff - 1],
                device_id=(d,), device_id_type=pl.DeviceIdType.MESH,
            )
            rdma.start()
            sends.append(rdma)

        for joff in range(1, N_DEV):
            o = (my - joff) % N_DEV
            recv = pltpu.make_async_remote_copy(
                src_ref=q_ref.at[pl.ds(0, m_per)],
                dst_ref=tiles_ref.at[pl.ds(o * m_per, m_per)],
                send_sem=t_ssems.at[joff - 1],
                recv_sem=t_rsems.at[joff - 1],
                device_id=(o,), device_id_type=pl.DeviceIdType.MESH,
            )
            recv.wait_recv()
            out_ref[pl.ds(o * m_per, m_per), :] = (
                tiles_ref[pl.ds(o * m_per, m_per), :].astype(jnp.float32)
                * scale
            )

        for rdma in sends:
            rdma.wait_send()

    return pl.pallas_call(
        body,
        out_shape=jax.ShapeDtypeStruct((m_total, n_per), jnp.float32),
        in_specs=[
            pl.BlockSpec(memory_space=pltpu.VMEM),
            pl.BlockSpec(memory_space=pltpu.VMEM),
        ],
        out_specs=pl.BlockSpec(memory_space=pltpu.VMEM),
        scratch_shapes=[
            pltpu.VMEM((N_DEV * KROWS[0], n_per), jnp.bfloat16),
            pltpu.VMEM((N_DEV * KROWS[1], n_per), jnp.bfloat16),
            pltpu.VMEM((N_DEV * KROWS[2], n_per), jnp.bfloat16),
            pltpu.SemaphoreType.DMA((3, 6)),
            pltpu.SemaphoreType.DMA((3, 6)),
            pltpu.VMEM((m_per, k), jnp.bfloat16),
            pltpu.VMEM((m_per, n_total), jnp.float32),
            pltpu.VMEM((N_DEV * m_per, n_per), jnp.int8),
            pltpu.VMEM((N_DEV * m_per, n_per), jnp.int8),
            pltpu.SemaphoreType.DMA((N_DEV - 1,)),
            pltpu.SemaphoreType.DMA((N_DEV - 1,)),
            pltpu.VMEM((4, 1, 128), jnp.float32),
            pltpu.SemaphoreType.DMA((3,)),
            pltpu.SemaphoreType.DMA((3,)),
        ],
        compiler_params=pltpu.CompilerParams(
            collective_id=0,
            vmem_limit_bytes=56 * 1024 * 1024,
        ),
    )(x, w_mat)


# device time: 87336 ns/iter; 1.5732x vs baseline; 1.0381x over previous
import jax
import jax.numpy as jnp
from jax import lax
from jax.experimental import pallas as pl
from jax.experimental.pallas import tpu as pltpu

N_DEV = 8
NBR = [
    [1, 0, 3, 2, 5, 4, 7, 6],
    [3, 2, 1, 0, 7, 6, 5, 4],
    [4, 5, 6, 7, 0, 1, 2, 3],
]
SLOT = [
    [0, 1, 3, 2, 4, 5, 7, 6],
    [0, 4, 5, 1, 2, 6, 7, 3],
    [0, 2, 6, 4, 1, 3, 7, 5],
]
OTAB = [
    [0, 1, 3, 2, 4, 5, 7, 6],
    [0, 3, 4, 7, 1, 2, 5, 6],
    [0, 4, 1, 5, 3, 7, 2, 6],
]
KROWS = [1376, 1360, 1360]
KOFF = [0, 1376, 2736]


def _sel(idx, table):
    acc = jnp.int32(0)
    for p, v in enumerate(table):
        acc = acc + jnp.where(idx == p, jnp.int32(v), 0)
    return acc


def kernel(x, w_mat):
    m_per, k = x.shape
    _, n_per = w_mat.shape
    m_total = N_DEV * m_per
    n_total = N_DEV * n_per

    def body(x_ref, w_ref, out_ref, b0, b1, b2, ssems, rsems,
             xb_ref, y_ref, q_ref, tiles_ref, t_ssems, t_rsems,
             amax_ref, a_ssems, a_rsems):
        my = lax.axis_index("i")
        bufs = [b0, b1, b2]
        nbrs = [_sel(my, NBR[d]) for d in range(3)]
        slots = [_sel(my, SLOT[r]) for r in range(3)]

        barrier_sem = pltpu.get_barrier_semaphore()
        for joff in range(1, N_DEV):
            pl.semaphore_signal(
                barrier_sem, inc=1,
                device_id=((my + joff) % N_DEV,),
                device_id_type=pl.DeviceIdType.MESH,
            )
        xb_ref[:, :] = x_ref[:, :].astype(jnp.bfloat16)
        y_ref[:, :] = jnp.zeros((m_per, n_total), jnp.float32)
        pl.semaphore_wait(barrier_sem, N_DEV - 1)

        def piece_matmul(r, s_dyn):
            o = _sel(s_dyn, OTAB[r])
            dot = jnp.dot(
                xb_ref[:, KOFF[r]:KOFF[r] + KROWS[r]],
                bufs[r][pl.ds(s_dyn * KROWS[r], KROWS[r]), :],
                preferred_element_type=jnp.float32,
            )
            y_ref[:, pl.ds(o * n_per, n_per)] = (
                y_ref[:, pl.ds(o * n_per, n_per)] + dot
            )

        for t in range(3):
            size = 1 << t
            waits = []
            for r in range(3):
                nbr = nbrs[(r + t) % 3]
                base = (slots[r] // size) * size
                if t == 0:
                    bufs[r][pl.ds(slots[r] * KROWS[r], KROWS[r]), :] = (
                        w_ref[KOFF[r]:KOFF[r] + KROWS[r], :]
                        .astype(jnp.bfloat16)
                    )
                if t < 2:
                    rdma = pltpu.make_async_remote_copy(
                        src_ref=bufs[r].at[pl.ds(base * KROWS[r], size * KROWS[r])],
                        dst_ref=bufs[r].at[pl.ds(base * KROWS[r], size * KROWS[r])],
                        send_sem=ssems.at[r, t], recv_sem=rsems.at[r, t],
                        device_id=(nbr,), device_id_type=pl.DeviceIdType.MESH,
                    )
                    rdma.start()
                    waits.append(rdma)
                else:
                    for j in range(4):
                        rdma = pltpu.make_async_remote_copy(
                            src_ref=bufs[r].at[pl.ds((base + j) * KROWS[r], KROWS[r])],
                            dst_ref=bufs[r].at[pl.ds((base + j) * KROWS[r], KROWS[r])],
                            send_sem=ssems.at[r, 2 + j], recv_sem=rsems.at[r, 2 + j],
                            device_id=(nbr,), device_id_type=pl.DeviceIdType.MESH,
                        )
                        rdma.start()
                        waits.append(rdma)

            if t == 0:
                for r in range(3):
                    piece_matmul(r, slots[r])
            elif t == 1:
                for r in range(3):
                    rs = slots[r] + 1 - 2 * (slots[r] % 2)
                    piece_matmul(r, rs)
            else:
                for r in range(3):
                    b1bit = (slots[r] // 2) % 2
                    rb = (slots[r] // 4) * 4 + (1 - b1bit) * 2
                    piece_matmul(r, rb)
                    piece_matmul(r, rb + 1)

            if t < 2:
                for rdma in waits:
                    rdma.wait()
            else:
                for j in range(4):
                    for r in range(3):
                        waits[r * 4 + j].wait()
                    for r in range(3):
                        rbase = (1 - slots[r] // 4) * 4
                        piece_matmul(r, rbase + j)

        local_amax = jnp.max(jnp.abs(y_ref[:, :]))
        amax_ref[pl.ds(my, 1), :, :] = jnp.full((1, 1, 128), local_amax,
                                                jnp.float32)
        amax_waits = []
        for joff in range(1, N_DEV):
            d = (my + joff) % N_DEV
            rdma = pltpu.make_async_remote_copy(
                src_ref=amax_ref.at[pl.ds(my, 1)],
                dst_ref=amax_ref.at[pl.ds(my, 1)],
                send_sem=a_ssems.at[joff - 1],
                recv_sem=a_rsems.at[joff - 1],
                device_id=(d,), device_id_type=pl.DeviceIdType.MESH,
            )
            rdma.start()
            amax_waits.append(rdma)
        for rdma in amax_waits:
            rdma.wait_recv()
        gmax = jnp.max(amax_ref[:, :, :])
        scale = gmax / 127.0
        inv = 127.0 / gmax

        sends = []
        for joff in range(1, N_DEV):
            d = (my + joff) % N_DEV
            yblk = y_ref[:, pl.ds(d * n_per, n_per)]
            q = jnp.clip(jnp.round(yblk * inv), -127.0, 127.0)
            q_ref[pl.ds(d * m_per, m_per), :] = q.astype(jnp.int8)
            rdma = pltpu.make_async_remote_copy(
                src_ref=q_ref.at[pl.ds(d * m_per, m_per)],
                dst_ref=tiles_ref.at[pl.ds(my * m_per, m_per)],
                send_sem=t_ssems.at[joff - 1],
                recv_sem=t_rsems.at[joff - 1],
                device_id=(d,), device_id_type=pl.DeviceIdType.MESH,
            )
            rdma.start()
            sends.append(rdma)

        y_own = y_ref[:, pl.ds(my * n_per, n_per)]
        q_own = jnp.clip(jnp.round(y_own * inv), -127.0, 127.0)
        out_ref[pl.ds(my * m_per, m_per), :] = q_own * scale

        for joff in range(1, N_DEV):
            o = (my - joff) % N_DEV
            recv = pltpu.make_async_remote_copy(
                src_ref=q_ref.at[pl.ds(0, m_per)],
                dst_ref=tiles_ref.at[pl.ds(o * m_per, m_per)],
                send_sem=t_ssems.at[joff - 1],
                recv_sem=t_rsems.at[joff - 1],
                device_id=(o,), device_id_type=pl.DeviceIdType.MESH,
            )
            recv.wait_recv()
            out_ref[pl.ds(o * m_per, m_per), :] = (
                tiles_ref[pl.ds(o * m_per, m_per), :].astype(jnp.float32)
                * scale
            )

        for rdma in sends:
            rdma.wait_send()
        for rdma in amax_waits:
            rdma.wait_send()

    return pl.pallas_call(
        body,
        out_shape=jax.ShapeDtypeStruct((m_total, n_per), jnp.float32),
        in_specs=[
            pl.BlockSpec(memory_space=pltpu.VMEM),
            pl.BlockSpec(memory_space=pltpu.VMEM),
        ],
        out_specs=pl.BlockSpec(memory_space=pltpu.VMEM),
        scratch_shapes=[
            pltpu.VMEM((N_DEV * KROWS[0], n_per), jnp.bfloat16),
            pltpu.VMEM((N_DEV * KROWS[1], n_per), jnp.bfloat16),
            pltpu.VMEM((N_DEV * KROWS[2], n_per), jnp.bfloat16),
            pltpu.SemaphoreType.DMA((3, 6)),
            pltpu.SemaphoreType.DMA((3, 6)),
            pltpu.VMEM((m_per, k), jnp.bfloat16),
            pltpu.VMEM((m_per, n_total), jnp.float32),
            pltpu.VMEM((N_DEV * m_per, n_per), jnp.int8),
            pltpu.VMEM((N_DEV * m_per, n_per), jnp.int8),
            pltpu.SemaphoreType.DMA((N_DEV - 1,)),
            pltpu.SemaphoreType.DMA((N_DEV - 1,)),
            pltpu.VMEM((N_DEV, 1, 128), jnp.float32),
            pltpu.SemaphoreType.DMA((N_DEV - 1,)),
            pltpu.SemaphoreType.DMA((N_DEV - 1,)),
        ],
        compiler_params=pltpu.CompilerParams(
            collective_id=0,
            vmem_limit_bytes=56 * 1024 * 1024,
        ),
    )(x, w_mat)
